# baseline (speedup 1.0000x reference)
import functools

import jax
import jax.numpy as jnp
from jax import lax
from jax.experimental import pallas as pl
from jax.experimental.pallas import tpu as pltpu
from jax.experimental.pallas import tpu_sc as plsc

N = 10000
NPAD = 10240
D = 128
DH = 64
G = 64
NC = 2
NS = 16
CHUNK = 128
ROW_BLK = 1000
N_BLKS = N // ROW_BLK


def _make_edge_agg(t_edges):
    n_chunks = t_edges // CHUNK
    rows_per_tile = NPAD // NS
    zero_rows = 128

    mesh = plsc.VectorSubcoreMesh(core_axis_name="c", subcore_axis_name="s")

    @functools.partial(
        pl.kernel,
        mesh=mesh,
        compiler_params=pltpu.CompilerParams(use_tc_tiling_on_sc=False),
        out_type=jax.ShapeDtypeStruct((NC * NPAD, DH), jnp.float32),
        scratch_types=(
            [pltpu.VMEM((8, CHUNK), jnp.int32) for _ in range(4)]
            + [pltpu.VMEM((8, CHUNK), jnp.float32) for _ in range(4)]
            + [pltpu.VMEM((CHUNK, DH), jnp.float32) for _ in range(2)]
            + [pltpu.VMEM_SHARED((NPAD, DH), jnp.float32)]
            + [pltpu.VMEM_SHARED((NPAD, DH), jnp.float32)]
            + [pltpu.SemaphoreType.DMA for _ in range(12)]
        ),
    )
    def edge_agg(hws0_hbm, hws1_hbm, pk_hbm, w_hbm, out_hbm,
                 pk0, pk1, pk2, pk3, wv0, wv1, wv2, wv3, rw0, rw1, hw_sp, acc,
                 ps0, ps1, ps2, ps3, ws0, ws1, ws2, ws3, gs0, gs1, ss0, ss1):
        pk = [pk0, pk1, pk2, pk3]
        wv = [wv0, wv1, wv2, wv3]
        rw = [rw0, rw1]
        ps = [ps0, ps1, ps2, ps3]
        ws = [ws0, ws1, ws2, ws3]
        gs = [gs0, gs1]
        ss = [ss0, ss1]
        cid = lax.axis_index("c")
        sid = lax.axis_index("s")
        cbase = sid * n_chunks

        r0 = sid * rows_per_tile

        @pl.when(cid == 0)
        def _():
            pltpu.sync_copy(hws0_hbm.at[pl.ds(r0, rows_per_tile)],
                            hw_sp.at[pl.ds(r0, rows_per_tile)])

        @pl.when(cid == 1)
        def _():
            pltpu.sync_copy(hws1_hbm.at[pl.ds(r0, rows_per_tile)],
                            hw_sp.at[pl.ds(r0, rows_per_tile)])

        def zrow(r, _):
            for k in range(DH // 16):
                rw0[r, pl.ds(k * 16, 16)] = jnp.zeros((16,), jnp.float32)
            return 0
        lax.fori_loop(0, CHUNK, zrow, 0)
        for j in range(rows_per_tile // zero_rows):
            pltpu.sync_copy(
                rw0.at[pl.ds(0, zero_rows)],
                acc.at[pl.ds(r0 + j * zero_rows, zero_rows)],
            )
        plsc.subcore_barrier()

        def scale_chunk(rows_v, w_v):
            def grp_scale(g, _):
                w16 = w_v[0, pl.ds(g * 16, 16)]
                for j in range(16):
                    ws = w16[j]
                    e = g * 16 + j
                    for k in range(DH // 16):
                        rows_v[e, pl.ds(k * 16, 16)] = (
                            rows_v[e, pl.ds(k * 16, 16)] * ws)
                return 0
            lax.fori_loop(0, CHUNK // 16, grp_scale, 0)

        pltpu.sync_copy(pk_hbm.at[cbase], pk[0])
        pltpu.sync_copy(w_hbm.at[cbase], wv[0])
        pltpu.async_copy(pk_hbm.at[cbase + 1], pk[1], ps[1])
        pltpu.async_copy(w_hbm.at[cbase + 1], wv[1], ws[1])
        pltpu.async_copy(hw_sp.at[pk[0].at[0]], rw[0], gs[0])

        def outer(i, _):
            c0 = i * 4
            for q in range(4):
                c = c0 + q
                b = q & 1
                fq = (q + 2) & 3
                nq = (q + 1) & 3

                pltpu.make_async_copy(
                    hw_sp.at[pk[q].at[0]], rw[b], gs[b]).wait()

                @pl.when(c >= 1)
                def _():
                    pltpu.make_async_copy(
                        w_hbm.at[cbase + c], wv[q], ws[q]).wait()
                scale_chunk(rw[b], wv[q])

                @pl.when(c >= 1)
                def _():
                    pltpu.make_async_copy(
                        rw[1 - b], acc.at[pl.ds(0, CHUNK)], ss[1 - b]).wait()

                @pl.when(c + 2 < n_chunks)
                def _():
                    pltpu.async_copy(pk_hbm.at[cbase + c + 2], pk[fq], ps[fq])
                    pltpu.async_copy(w_hbm.at[cbase + c + 2], wv[fq], ws[fq])

                @pl.when(c + 1 < n_chunks)
                def _():
                    pltpu.make_async_copy(
                        pk_hbm.at[cbase + c + 1], pk[nq], ps[nq]).wait()
                    pltpu.async_copy(
                        hw_sp.at[pk[nq].at[0]], rw[1 - b], gs[1 - b])

                pltpu.async_copy(rw[b], acc.at[pk[q].at[1]], ss[b], add=True)
            return 0
        lax.fori_loop(0, n_chunks // 4, outer, 0)

        qlast = (n_chunks - 1) & 1
        pltpu.make_async_copy(
            rw[qlast], acc.at[pl.ds(0, CHUNK)], ss[qlast]).wait()

        plsc.subcore_barrier()

        pltpu.sync_copy(
            acc.at[pl.ds(r0, rows_per_tile)],
            out_hbm.at[pl.ds(cid * NPAD + r0, rows_per_tile)],
        )

    return edge_agg


def _split_out(res, o_ref, h0_ref, h1_ref):
    o_ref[...] = res
    h0_ref[...] = res[:, :DH]
    h1_ref[...] = res[:, DH:]


def _mm_kernel(x_ref, w_ref, o_ref, h0_ref, h1_ref):
    res = jnp.dot(x_ref[...], w_ref[...], preferred_element_type=jnp.float32)
    _split_out(res, o_ref, h0_ref, h1_ref)


_OUT3 = [
    jax.ShapeDtypeStruct((N, D), jnp.float32),
    jax.ShapeDtypeStruct((NPAD, DH), jnp.float32),
    jax.ShapeDtypeStruct((NPAD, DH), jnp.float32),
]
_OUT3_SPECS = [
    pl.BlockSpec((ROW_BLK, D), lambda i: (i, 0)),
    pl.BlockSpec((ROW_BLK, DH), lambda i: (i, 0)),
    pl.BlockSpec((ROW_BLK, DH), lambda i: (i, 0)),
]


def _tc_matmul(x, w):
    return pl.pallas_call(
        _mm_kernel,
        grid=(N_BLKS,),
        in_specs=[
            pl.BlockSpec((ROW_BLK, D), lambda i: (i, 0)),
            pl.BlockSpec((D, D), lambda i: (0, 0)),
        ],
        out_specs=_OUT3_SPECS,
        out_shape=_OUT3,
    )(x, w)


def _fused_kernel(z0_ref, z1_ref, hw_ref, b_ref, w_ref, o_ref, h0_ref, h1_ref):
    z = jnp.concatenate([z0_ref[...], z1_ref[...]], axis=1)
    h = jax.nn.relu(z + hw_ref[...] + b_ref[...])
    res = jnp.dot(h, w_ref[...], preferred_element_type=jnp.float32)
    _split_out(res, o_ref, h0_ref, h1_ref)


def _tc_fused_layer(z0, z1, hw, b, w):
    return pl.pallas_call(
        _fused_kernel,
        grid=(N_BLKS,),
        in_specs=[
            pl.BlockSpec((ROW_BLK, DH), lambda i: (i, 0)),
            pl.BlockSpec((ROW_BLK, DH), lambda i: (i, 0)),
            pl.BlockSpec((ROW_BLK, D), lambda i: (i, 0)),
            pl.BlockSpec((1, D), lambda i: (0, 0)),
            pl.BlockSpec((D, D), lambda i: (0, 0)),
        ],
        out_specs=_OUT3_SPECS,
        out_shape=_OUT3,
    )(z0, z1, hw, b, w)


def _pool_kernel(z0_ref, z1_ref, hw_ref, b_ref, batch_ref, w4_ref, b4_ref,
                 o_ref, sums_ref, cnts_ref):
    i = pl.program_id(0)

    @pl.when(i == 0)
    def _():
        sums_ref[...] = jnp.zeros_like(sums_ref)
        cnts_ref[...] = jnp.zeros_like(cnts_ref)

    z = jnp.concatenate([z0_ref[...], z1_ref[...]], axis=1)
    h = jax.nn.relu(z + hw_ref[...] + b_ref[...])
    bids = batch_ref[0]
    gids = lax.broadcasted_iota(jnp.int32, (G, ROW_BLK), 0)
    mask = (bids == gids).astype(jnp.float32)
    sums_ref[...] += jnp.dot(mask, h, preferred_element_type=jnp.float32)
    cnts_ref[...] += jnp.sum(mask, axis=1, keepdims=True)

    @pl.when(i == N_BLKS - 1)
    def _():
        pooled = sums_ref[...] / jnp.maximum(cnts_ref[...], 1.0)
        o_ref[...] = jnp.dot(pooled, w4_ref[...],
                             preferred_element_type=jnp.float32) + b4_ref[...]


def _tc_pool(z0, z1, hw, b, batch3d, w4, b4):
    return pl.pallas_call(
        _pool_kernel,
        grid=(N_BLKS,),
        in_specs=[
            pl.BlockSpec((ROW_BLK, DH), lambda i: (i, 0)),
            pl.BlockSpec((ROW_BLK, DH), lambda i: (i, 0)),
            pl.BlockSpec((ROW_BLK, D), lambda i: (i, 0)),
            pl.BlockSpec((1, D), lambda i: (0, 0)),
            pl.BlockSpec((1, 1, ROW_BLK), lambda i: (i, 0, 0)),
            pl.BlockSpec((D, D), lambda i: (0, 0)),
            pl.BlockSpec((1, D), lambda i: (0, 0)),
        ],
        out_specs=pl.BlockSpec((G, D), lambda i: (0, 0)),
        out_shape=jax.ShapeDtypeStruct((G, D), jnp.float32),
        scratch_shapes=[
            pltpu.VMEM((G, D), jnp.float32),
            pltpu.VMEM((G, D), jnp.float32),
        ],
    )(z0, z1, hw, b, batch3d, w4, b4)


def kernel(x, edge_index, batch, edge_weights, W1, b1, W2, b2, W3, b3, W4, b4):
    E = edge_index.shape[1]
    t_edges = -(-E // (NS * CHUNK * 4)) * CHUNK * 4
    e_pad = NS * t_edges

    src = edge_index[0].astype(jnp.int32)
    dst = edge_index[1].astype(jnp.int32)
    w = edge_weights.astype(jnp.float32)
    pad = e_pad - E
    if pad:
        src = jnp.concatenate([src, jnp.zeros((pad,), jnp.int32)])
        dst = jnp.concatenate([dst, jnp.zeros((pad,), jnp.int32)])
        w = jnp.concatenate([w, jnp.zeros((pad,), jnp.float32)])

    n_all_chunks = e_pad // CHUNK
    pk = jnp.concatenate(
        [
            jnp.stack(
                [src.reshape(n_all_chunks, CHUNK),
                 dst.reshape(n_all_chunks, CHUNK)],
                axis=1,
            ),
            jnp.zeros((n_all_chunks, 6, CHUNK), jnp.int32),
        ],
        axis=1,
    )
    w8 = jnp.concatenate(
        [
            w.reshape(n_all_chunks, 1, CHUNK),
            jnp.zeros((n_all_chunks, 7, CHUNK), jnp.float32),
        ],
        axis=1,
    )

    edge_agg = _make_edge_agg(t_edges)

    b1r = b1.reshape(1, D)
    b2r = b2.reshape(1, D)
    b3r = b3.reshape(1, D)
    b4r = b4.reshape(1, D)
    batch3d = batch.astype(jnp.int32).reshape(N_BLKS, 1, ROW_BLK)

    hw1, h10, h11 = _tc_matmul(x, W1)
    z1 = edge_agg(h10, h11, pk, w8)
    hw2, h20, h21 = _tc_fused_layer(z1[:N], z1[NPAD:NPAD + N], hw1, b1r, W2)
    z2 = edge_agg(h20, h21, pk, w8)
    hw3, h30, h31 = _tc_fused_layer(z2[:N], z2[NPAD:NPAD + N], hw2, b2r, W3)
    z3 = edge_agg(h30, h31, pk, w8)
    return _tc_pool(z3[:N], z3[NPAD:NPAD + N], hw3, b3r, batch3d, W4, b4r)


# speedup vs baseline: 1.1771x; 1.1771x over previous
import functools

import jax
import jax.numpy as jnp
from jax import lax
from jax.experimental import pallas as pl
from jax.experimental.pallas import tpu as pltpu
from jax.experimental.pallas import tpu_sc as plsc

N = 10000
NPAD = 10240
D = 128
DH = 64
G = 64
NC = 2
NS = 16
CHUNK = 128
ROW_BLK = 1000
N_BLKS = N // ROW_BLK


def _make_edge_agg(t_edges):
    n_chunks = t_edges // CHUNK
    rows_per_tile = NPAD // NS
    zero_rows = 128

    mesh = plsc.VectorSubcoreMesh(core_axis_name="c", subcore_axis_name="s")

    @functools.partial(
        pl.kernel,
        mesh=mesh,
        compiler_params=pltpu.CompilerParams(use_tc_tiling_on_sc=False),
        out_type=jax.ShapeDtypeStruct((NC * NPAD, DH), jnp.float32),
        scratch_types=(
            [pltpu.VMEM((8, CHUNK), jnp.int32) for _ in range(8)]
            + [pltpu.VMEM((8, CHUNK), jnp.float32) for _ in range(4)]
            + [pltpu.VMEM((CHUNK, DH), jnp.float32) for _ in range(4)]
            + [pltpu.VMEM_SHARED((NPAD, DH), jnp.float32)]
            + [pltpu.VMEM_SHARED((NPAD, DH), jnp.float32)]
            + [pltpu.SemaphoreType.DMA for _ in range(20)]
        ),
    )
    def edge_agg(hws0_hbm, hws1_hbm, pk_hbm, w_hbm, out_hbm,
                 pk0, pk1, pk2, pk3, pk4, pk5, pk6, pk7,
                 wv0, wv1, wv2, wv3, rw0, rw1, rw2, rw3, hw_sp, acc,
                 ps0, ps1, ps2, ps3, ps4, ps5, ps6, ps7,
                 ws0, ws1, ws2, ws3, gs0, gs1, gs2, gs3, ss0, ss1, ss2, ss3):
        pk = [pk0, pk1, pk2, pk3, pk4, pk5, pk6, pk7]
        wv = [wv0, wv1, wv2, wv3]
        rw = [rw0, rw1, rw2, rw3]
        ps = [ps0, ps1, ps2, ps3, ps4, ps5, ps6, ps7]
        ws = [ws0, ws1, ws2, ws3]
        gs = [gs0, gs1, gs2, gs3]
        ss = [ss0, ss1, ss2, ss3]
        cid = lax.axis_index("c")
        sid = lax.axis_index("s")
        cbase = sid * n_chunks

        r0 = sid * rows_per_tile

        @pl.when(cid == 0)
        def _():
            pltpu.sync_copy(hws0_hbm.at[pl.ds(r0, rows_per_tile)],
                            hw_sp.at[pl.ds(r0, rows_per_tile)])

        @pl.when(cid == 1)
        def _():
            pltpu.sync_copy(hws1_hbm.at[pl.ds(r0, rows_per_tile)],
                            hw_sp.at[pl.ds(r0, rows_per_tile)])

        def zrow(r, _):
            for k in range(DH // 16):
                rw0[r, pl.ds(k * 16, 16)] = jnp.zeros((16,), jnp.float32)
            return 0
        lax.fori_loop(0, CHUNK, zrow, 0)
        for j in range(rows_per_tile // zero_rows):
            pltpu.sync_copy(
                rw0.at[pl.ds(0, zero_rows)],
                acc.at[pl.ds(r0 + j * zero_rows, zero_rows)],
            )
        plsc.subcore_barrier()

        def scale_chunk(rows_v, w_v):
            def grp_scale(g, _):
                w16 = w_v[0, pl.ds(g * 16, 16)]
                for j in range(16):
                    ws = w16[j]
                    e = g * 16 + j
                    for k in range(DH // 16):
                        rows_v[e, pl.ds(k * 16, 16)] = (
                            rows_v[e, pl.ds(k * 16, 16)] * ws)
                return 0
            lax.fori_loop(0, CHUNK // 16, grp_scale, 0)

        pltpu.sync_copy(pk_hbm.at[cbase], pk[0])
        pltpu.sync_copy(pk_hbm.at[cbase + 1], pk[1])
        pltpu.async_copy(pk_hbm.at[cbase + 2], pk[2], ps[2])
        pltpu.sync_copy(w_hbm.at[cbase], wv[0])
        pltpu.async_copy(w_hbm.at[cbase + 1], wv[1], ws[1])
        pltpu.async_copy(w_hbm.at[cbase + 2], wv[2], ws[2])
        pltpu.async_copy(hw_sp.at[pk[0].at[0]], rw[0], gs[0])
        pltpu.async_copy(hw_sp.at[pk[1].at[0]], rw[1], gs[1])

        def outer(i, _):
            c0 = i * 8
            for p in range(8):
                c = c0 + p
                q = p & 3
                f3 = (p + 3) & 7
                f2 = (p + 2) & 7
                q3 = (q + 3) & 3
                q2 = (q + 2) & 3

                @pl.when(c >= 2)
                def _():
                    pltpu.make_async_copy(
                        rw[q2], acc.at[pl.ds(0, CHUNK)], ss[q2]).wait()

                @pl.when(c + 3 < n_chunks)
                def _():
                    pltpu.async_copy(pk_hbm.at[cbase + c + 3], pk[f3], ps[f3])
                    pltpu.async_copy(w_hbm.at[cbase + c + 3], wv[q3], ws[q3])

                @pl.when(c + 2 < n_chunks)
                def _():
                    pltpu.make_async_copy(
                        pk_hbm.at[cbase + c + 2], pk[f2], ps[f2]).wait()
                    pltpu.async_copy(
                        hw_sp.at[pk[f2].at[0]], rw[q2], gs[q2])

                pltpu.make_async_copy(
                    hw_sp.at[pk[p].at[0]], rw[q], gs[q]).wait()

                @pl.when(c >= 1)
                def _():
                    pltpu.make_async_copy(
                        w_hbm.at[cbase + c], wv[q], ws[q]).wait()
                scale_chunk(rw[q], wv[q])

                pltpu.async_copy(rw[q], acc.at[pk[p].at[1]], ss[q], add=True)
            return 0
        lax.fori_loop(0, n_chunks // 8, outer, 0)

        for c in (n_chunks - 2, n_chunks - 1):
            pltpu.make_async_copy(
                rw[c & 3], acc.at[pl.ds(0, CHUNK)], ss[c & 3]).wait()

        plsc.subcore_barrier()

        pltpu.sync_copy(
            acc.at[pl.ds(r0, rows_per_tile)],
            out_hbm.at[pl.ds(cid * NPAD + r0, rows_per_tile)],
        )

    return edge_agg


def _split_out(res, o_ref, h0_ref, h1_ref):
    o_ref[...] = res
    h0_ref[...] = res[:, :DH]
    h1_ref[...] = res[:, DH:]


def _mm_kernel(x_ref, w_ref, o_ref, h0_ref, h1_ref):
    res = jnp.dot(x_ref[...], w_ref[...], preferred_element_type=jnp.float32)
    _split_out(res, o_ref, h0_ref, h1_ref)


_OUT3 = [
    jax.ShapeDtypeStruct((N, D), jnp.float32),
    jax.ShapeDtypeStruct((NPAD, DH), jnp.float32),
    jax.ShapeDtypeStruct((NPAD, DH), jnp.float32),
]
_OUT3_SPECS = [
    pl.BlockSpec((ROW_BLK, D), lambda i: (i, 0)),
    pl.BlockSpec((ROW_BLK, DH), lambda i: (i, 0)),
    pl.BlockSpec((ROW_BLK, DH), lambda i: (i, 0)),
]


def _tc_matmul(x, w):
    return pl.pallas_call(
        _mm_kernel,
        grid=(N_BLKS,),
        in_specs=[
            pl.BlockSpec((ROW_BLK, D), lambda i: (i, 0)),
            pl.BlockSpec((D, D), lambda i: (0, 0)),
        ],
        out_specs=_OUT3_SPECS,
        out_shape=_OUT3,
    )(x, w)


def _fused_kernel(z0_ref, z1_ref, hw_ref, b_ref, w_ref, o_ref, h0_ref, h1_ref):
    z = jnp.concatenate([z0_ref[...], z1_ref[...]], axis=1)
    h = jax.nn.relu(z + hw_ref[...] + b_ref[...])
    res = jnp.dot(h, w_ref[...], preferred_element_type=jnp.float32)
    _split_out(res, o_ref, h0_ref, h1_ref)


def _tc_fused_layer(z0, z1, hw, b, w):
    return pl.pallas_call(
        _fused_kernel,
        grid=(N_BLKS,),
        in_specs=[
            pl.BlockSpec((ROW_BLK, DH), lambda i: (i, 0)),
            pl.BlockSpec((ROW_BLK, DH), lambda i: (i, 0)),
            pl.BlockSpec((ROW_BLK, D), lambda i: (i, 0)),
            pl.BlockSpec((1, D), lambda i: (0, 0)),
            pl.BlockSpec((D, D), lambda i: (0, 0)),
        ],
        out_specs=_OUT3_SPECS,
        out_shape=_OUT3,
    )(z0, z1, hw, b, w)


def _pool_kernel(z0_ref, z1_ref, hw_ref, b_ref, batch_ref, w4_ref, b4_ref,
                 o_ref, sums_ref, cnts_ref):
    i = pl.program_id(0)

    @pl.when(i == 0)
    def _():
        sums_ref[...] = jnp.zeros_like(sums_ref)
        cnts_ref[...] = jnp.zeros_like(cnts_ref)

    z = jnp.concatenate([z0_ref[...], z1_ref[...]], axis=1)
    h = jax.nn.relu(z + hw_ref[...] + b_ref[...])
    bids = batch_ref[0]
    gids = lax.broadcasted_iota(jnp.int32, (G, ROW_BLK), 0)
    mask = (bids == gids).astype(jnp.float32)
    sums_ref[...] += jnp.dot(mask, h, preferred_element_type=jnp.float32)
    cnts_ref[...] += jnp.sum(mask, axis=1, keepdims=True)

    @pl.when(i == N_BLKS - 1)
    def _():
        pooled = sums_ref[...] / jnp.maximum(cnts_ref[...], 1.0)
        o_ref[...] = jnp.dot(pooled, w4_ref[...],
                             preferred_element_type=jnp.float32) + b4_ref[...]


def _tc_pool(z0, z1, hw, b, batch3d, w4, b4):
    return pl.pallas_call(
        _pool_kernel,
        grid=(N_BLKS,),
        in_specs=[
            pl.BlockSpec((ROW_BLK, DH), lambda i: (i, 0)),
            pl.BlockSpec((ROW_BLK, DH), lambda i: (i, 0)),
            pl.BlockSpec((ROW_BLK, D), lambda i: (i, 0)),
            pl.BlockSpec((1, D), lambda i: (0, 0)),
            pl.BlockSpec((1, 1, ROW_BLK), lambda i: (i, 0, 0)),
            pl.BlockSpec((D, D), lambda i: (0, 0)),
            pl.BlockSpec((1, D), lambda i: (0, 0)),
        ],
        out_specs=pl.BlockSpec((G, D), lambda i: (0, 0)),
        out_shape=jax.ShapeDtypeStruct((G, D), jnp.float32),
        scratch_shapes=[
            pltpu.VMEM((G, D), jnp.float32),
            pltpu.VMEM((G, D), jnp.float32),
        ],
    )(z0, z1, hw, b, batch3d, w4, b4)


def kernel(x, edge_index, batch, edge_weights, W1, b1, W2, b2, W3, b3, W4, b4):
    E = edge_index.shape[1]
    t_edges = -(-E // (NS * CHUNK * 8)) * CHUNK * 8
    e_pad = NS * t_edges

    src = edge_index[0].astype(jnp.int32)
    dst = edge_index[1].astype(jnp.int32)
    w = edge_weights.astype(jnp.float32)
    pad = e_pad - E
    if pad:
        src = jnp.concatenate([src, jnp.zeros((pad,), jnp.int32)])
        dst = jnp.concatenate([dst, jnp.zeros((pad,), jnp.int32)])
        w = jnp.concatenate([w, jnp.zeros((pad,), jnp.float32)])

    n_all_chunks = e_pad // CHUNK
    pk = jnp.concatenate(
        [
            jnp.stack(
                [src.reshape(n_all_chunks, CHUNK),
                 dst.reshape(n_all_chunks, CHUNK)],
                axis=1,
            ),
            jnp.zeros((n_all_chunks, 6, CHUNK), jnp.int32),
        ],
        axis=1,
    )
    w8 = jnp.concatenate(
        [
            w.reshape(n_all_chunks, 1, CHUNK),
            jnp.zeros((n_all_chunks, 7, CHUNK), jnp.float32),
        ],
        axis=1,
    )

    edge_agg = _make_edge_agg(t_edges)

    b1r = b1.reshape(1, D)
    b2r = b2.reshape(1, D)
    b3r = b3.reshape(1, D)
    b4r = b4.reshape(1, D)
    batch3d = batch.astype(jnp.int32).reshape(N_BLKS, 1, ROW_BLK)

    hw1, h10, h11 = _tc_matmul(x, W1)
    z1 = edge_agg(h10, h11, pk, w8)
    hw2, h20, h21 = _tc_fused_layer(z1[:N], z1[NPAD:NPAD + N], hw1, b1r, W2)
    z2 = edge_agg(h20, h21, pk, w8)
    hw3, h30, h31 = _tc_fused_layer(z2[:N], z2[NPAD:NPAD + N], hw2, b2r, W3)
    z3 = edge_agg(h30, h31, pk, w8)
    return _tc_pool(z3[:N], z3[NPAD:NPAD + N], hw3, b3r, batch3d, W4, b4r)


# speedup vs baseline: 2.6624x; 2.2618x over previous
import functools

import jax
import jax.numpy as jnp
from jax import lax
from jax.experimental import pallas as pl
from jax.experimental.pallas import tpu as pltpu
from jax.experimental.pallas import tpu_sc as plsc

N = 10000
NPAD = 10240
D = 128
DH = 64
G = 64
NC = 2
NS = 16
CHUNK = 128
ROW_BLK = 1000
N_BLKS = N // ROW_BLK


def _make_edge_agg(t_edges):
    n_chunks = t_edges // CHUNK
    rows_per_tile = NPAD // NS
    zero_rows = 128

    mesh = plsc.VectorSubcoreMesh(core_axis_name="c", subcore_axis_name="s")

    @functools.partial(
        pl.kernel,
        mesh=mesh,
        compiler_params=pltpu.CompilerParams(use_tc_tiling_on_sc=False),
        out_type=jax.ShapeDtypeStruct((NC * NPAD, DH), jnp.float32),
        scratch_types=(
            [pltpu.VMEM((8, CHUNK), jnp.int32) for _ in range(8)]
            + [pltpu.VMEM((8, CHUNK), jnp.float32) for _ in range(4)]
            + [pltpu.VMEM((CHUNK, DH), jnp.float32) for _ in range(4)]
            + [pltpu.VMEM_SHARED((NPAD, DH), jnp.float32)]
            + [pltpu.VMEM_SHARED((NPAD, DH), jnp.float32)]
            + [pltpu.SemaphoreType.DMA for _ in range(20)]
        ),
    )
    def edge_agg(hws0_hbm, hws1_hbm, pk_hbm, w_hbm, out_hbm,
                 pk0, pk1, pk2, pk3, pk4, pk5, pk6, pk7,
                 wv0, wv1, wv2, wv3, rw0, rw1, rw2, rw3, hw_sp, acc,
                 ps0, ps1, ps2, ps3, ps4, ps5, ps6, ps7,
                 ws0, ws1, ws2, ws3, gs0, gs1, gs2, gs3, ss0, ss1, ss2, ss3):
        pk = [pk0, pk1, pk2, pk3, pk4, pk5, pk6, pk7]
        wv = [wv0, wv1, wv2, wv3]
        rw = [rw0, rw1, rw2, rw3]
        ps = [ps0, ps1, ps2, ps3, ps4, ps5, ps6, ps7]
        ws = [ws0, ws1, ws2, ws3]
        gs = [gs0, gs1, gs2, gs3]
        ss = [ss0, ss1, ss2, ss3]
        cid = lax.axis_index("c")
        sid = lax.axis_index("s")
        cbase = sid * n_chunks

        r0 = sid * rows_per_tile

        @pl.when(cid == 0)
        def _():
            pltpu.sync_copy(hws0_hbm.at[pl.ds(r0, rows_per_tile)],
                            hw_sp.at[pl.ds(r0, rows_per_tile)])

        @pl.when(cid == 1)
        def _():
            pltpu.sync_copy(hws1_hbm.at[pl.ds(r0, rows_per_tile)],
                            hw_sp.at[pl.ds(r0, rows_per_tile)])

        def zrow(r, _):
            for k in range(DH // 16):
                rw0[r, pl.ds(k * 16, 16)] = jnp.zeros((16,), jnp.float32)
            return 0
        lax.fori_loop(0, CHUNK, zrow, 0)
        for j in range(rows_per_tile // zero_rows):
            pltpu.sync_copy(
                rw0.at[pl.ds(0, zero_rows)],
                acc.at[pl.ds(r0 + j * zero_rows, zero_rows)],
            )
        plsc.subcore_barrier()

        def scale_chunk(rows_v, w_v):
            def grp_scale(g, _):
                w16 = w_v[0, pl.ds(g * 16, 16)]
                for j in range(16):
                    ws = w16[j]
                    e = g * 16 + j
                    for k in range(DH // 16):
                        rows_v[e, pl.ds(k * 16, 16)] = (
                            rows_v[e, pl.ds(k * 16, 16)] * ws)
                return 0
            lax.fori_loop(0, CHUNK // 16, grp_scale, 0)

        pltpu.sync_copy(pk_hbm.at[cbase], pk[0])
        pltpu.sync_copy(pk_hbm.at[cbase + 1], pk[1])
        pltpu.async_copy(pk_hbm.at[cbase + 2], pk[2], ps[2])
        pltpu.sync_copy(w_hbm.at[cbase], wv[0])
        pltpu.async_copy(w_hbm.at[cbase + 1], wv[1], ws[1])
        pltpu.async_copy(w_hbm.at[cbase + 2], wv[2], ws[2])
        pltpu.async_copy(hw_sp.at[pk[0].at[0]], rw[0], gs[0])
        pltpu.async_copy(hw_sp.at[pk[1].at[0]], rw[1], gs[1])

        def outer(i, _):
            c0 = i * 8
            for p in range(8):
                c = c0 + p
                q = p & 3
                f3 = (p + 3) & 7
                f2 = (p + 2) & 7
                q3 = (q + 3) & 3
                q2 = (q + 2) & 3

                @pl.when(c >= 2)
                def _():
                    pltpu.make_async_copy(
                        rw[q2], acc.at[pl.ds(0, CHUNK)], ss[q2]).wait()

                @pl.when(c + 3 < n_chunks)
                def _():
                    pltpu.async_copy(pk_hbm.at[cbase + c + 3], pk[f3], ps[f3])
                    pltpu.async_copy(w_hbm.at[cbase + c + 3], wv[q3], ws[q3])

                @pl.when(c + 2 < n_chunks)
                def _():
                    pltpu.make_async_copy(
                        pk_hbm.at[cbase + c + 2], pk[f2], ps[f2]).wait()
                    pltpu.async_copy(
                        hw_sp.at[pk[f2].at[0]], rw[q2], gs[q2])

                pltpu.make_async_copy(
                    hw_sp.at[pk[p].at[0]], rw[q], gs[q]).wait()

                @pl.when(c >= 1)
                def _():
                    pltpu.make_async_copy(
                        w_hbm.at[cbase + c], wv[q], ws[q]).wait()

                pltpu.async_copy(rw[q], acc.at[pk[p].at[1]], ss[q], add=True)
            return 0
        lax.fori_loop(0, n_chunks // 8, outer, 0)

        for c in (n_chunks - 2, n_chunks - 1):
            pltpu.make_async_copy(
                rw[c & 3], acc.at[pl.ds(0, CHUNK)], ss[c & 3]).wait()

        plsc.subcore_barrier()

        pltpu.sync_copy(
            acc.at[pl.ds(r0, rows_per_tile)],
            out_hbm.at[pl.ds(cid * NPAD + r0, rows_per_tile)],
        )

    return edge_agg


def _split_out(res, o_ref, h0_ref, h1_ref):
    o_ref[...] = res
    h0_ref[...] = res[:, :DH]
    h1_ref[...] = res[:, DH:]


def _mm_kernel(x_ref, w_ref, o_ref, h0_ref, h1_ref):
    res = jnp.dot(x_ref[...], w_ref[...], preferred_element_type=jnp.float32)
    _split_out(res, o_ref, h0_ref, h1_ref)


_OUT3 = [
    jax.ShapeDtypeStruct((N, D), jnp.float32),
    jax.ShapeDtypeStruct((NPAD, DH), jnp.float32),
    jax.ShapeDtypeStruct((NPAD, DH), jnp.float32),
]
_OUT3_SPECS = [
    pl.BlockSpec((ROW_BLK, D), lambda i: (i, 0)),
    pl.BlockSpec((ROW_BLK, DH), lambda i: (i, 0)),
    pl.BlockSpec((ROW_BLK, DH), lambda i: (i, 0)),
]


def _tc_matmul(x, w):
    return pl.pallas_call(
        _mm_kernel,
        grid=(N_BLKS,),
        in_specs=[
            pl.BlockSpec((ROW_BLK, D), lambda i: (i, 0)),
            pl.BlockSpec((D, D), lambda i: (0, 0)),
        ],
        out_specs=_OUT3_SPECS,
        out_shape=_OUT3,
    )(x, w)


def _fused_kernel(z0_ref, z1_ref, hw_ref, b_ref, w_ref, o_ref, h0_ref, h1_ref):
    z = jnp.concatenate([z0_ref[...], z1_ref[...]], axis=1)
    h = jax.nn.relu(z + hw_ref[...] + b_ref[...])
    res = jnp.dot(h, w_ref[...], preferred_element_type=jnp.float32)
    _split_out(res, o_ref, h0_ref, h1_ref)


def _tc_fused_layer(z0, z1, hw, b, w):
    return pl.pallas_call(
        _fused_kernel,
        grid=(N_BLKS,),
        in_specs=[
            pl.BlockSpec((ROW_BLK, DH), lambda i: (i, 0)),
            pl.BlockSpec((ROW_BLK, DH), lambda i: (i, 0)),
            pl.BlockSpec((ROW_BLK, D), lambda i: (i, 0)),
            pl.BlockSpec((1, D), lambda i: (0, 0)),
            pl.BlockSpec((D, D), lambda i: (0, 0)),
        ],
        out_specs=_OUT3_SPECS,
        out_shape=_OUT3,
    )(z0, z1, hw, b, w)


def _pool_kernel(z0_ref, z1_ref, hw_ref, b_ref, batch_ref, w4_ref, b4_ref,
                 o_ref, sums_ref, cnts_ref):
    i = pl.program_id(0)

    @pl.when(i == 0)
    def _():
        sums_ref[...] = jnp.zeros_like(sums_ref)
        cnts_ref[...] = jnp.zeros_like(cnts_ref)

    z = jnp.concatenate([z0_ref[...], z1_ref[...]], axis=1)
    h = jax.nn.relu(z + hw_ref[...] + b_ref[...])
    bids = batch_ref[0]
    gids = lax.broadcasted_iota(jnp.int32, (G, ROW_BLK), 0)
    mask = (bids == gids).astype(jnp.float32)
    sums_ref[...] += jnp.dot(mask, h, preferred_element_type=jnp.float32)
    cnts_ref[...] += jnp.sum(mask, axis=1, keepdims=True)

    @pl.when(i == N_BLKS - 1)
    def _():
        pooled = sums_ref[...] / jnp.maximum(cnts_ref[...], 1.0)
        o_ref[...] = jnp.dot(pooled, w4_ref[...],
                             preferred_element_type=jnp.float32) + b4_ref[...]


def _tc_pool(z0, z1, hw, b, batch3d, w4, b4):
    return pl.pallas_call(
        _pool_kernel,
        grid=(N_BLKS,),
        in_specs=[
            pl.BlockSpec((ROW_BLK, DH), lambda i: (i, 0)),
            pl.BlockSpec((ROW_BLK, DH), lambda i: (i, 0)),
            pl.BlockSpec((ROW_BLK, D), lambda i: (i, 0)),
            pl.BlockSpec((1, D), lambda i: (0, 0)),
            pl.BlockSpec((1, 1, ROW_BLK), lambda i: (i, 0, 0)),
            pl.BlockSpec((D, D), lambda i: (0, 0)),
            pl.BlockSpec((1, D), lambda i: (0, 0)),
        ],
        out_specs=pl.BlockSpec((G, D), lambda i: (0, 0)),
        out_shape=jax.ShapeDtypeStruct((G, D), jnp.float32),
        scratch_shapes=[
            pltpu.VMEM((G, D), jnp.float32),
            pltpu.VMEM((G, D), jnp.float32),
        ],
    )(z0, z1, hw, b, batch3d, w4, b4)


def kernel(x, edge_index, batch, edge_weights, W1, b1, W2, b2, W3, b3, W4, b4):
    E = edge_index.shape[1]
    t_edges = -(-E // (NS * CHUNK * 8)) * CHUNK * 8
    e_pad = NS * t_edges

    src = edge_index[0].astype(jnp.int32)
    dst = edge_index[1].astype(jnp.int32)
    w = edge_weights.astype(jnp.float32)
    pad = e_pad - E
    if pad:
        src = jnp.concatenate([src, jnp.zeros((pad,), jnp.int32)])
        dst = jnp.concatenate([dst, jnp.zeros((pad,), jnp.int32)])
        w = jnp.concatenate([w, jnp.zeros((pad,), jnp.float32)])

    n_all_chunks = e_pad // CHUNK
    pk = jnp.concatenate(
        [
            jnp.stack(
                [src.reshape(n_all_chunks, CHUNK),
                 dst.reshape(n_all_chunks, CHUNK)],
                axis=1,
            ),
            jnp.zeros((n_all_chunks, 6, CHUNK), jnp.int32),
        ],
        axis=1,
    )
    w8 = jnp.concatenate(
        [
            w.reshape(n_all_chunks, 1, CHUNK),
            jnp.zeros((n_all_chunks, 7, CHUNK), jnp.float32),
        ],
        axis=1,
    )

    edge_agg = _make_edge_agg(t_edges)

    b1r = b1.reshape(1, D)
    b2r = b2.reshape(1, D)
    b3r = b3.reshape(1, D)
    b4r = b4.reshape(1, D)
    batch3d = batch.astype(jnp.int32).reshape(N_BLKS, 1, ROW_BLK)

    hw1, h10, h11 = _tc_matmul(x, W1)
    z1 = edge_agg(h10, h11, pk, w8)
    hw2, h20, h21 = _tc_fused_layer(z1[:N], z1[NPAD:NPAD + N], hw1, b1r, W2)
    z2 = edge_agg(h20, h21, pk, w8)
    hw3, h30, h31 = _tc_fused_layer(z2[:N], z2[NPAD:NPAD + N], hw2, b2r, W3)
    z3 = edge_agg(h30, h31, pk, w8)
    return _tc_pool(z3[:N], z3[NPAD:NPAD + N], hw3, b3r, batch3d, W4, b4r)
